# Initial kernel scaffold; baseline (speedup 1.0000x reference)
import jax
import jax.numpy as jnp
from jax.experimental import pallas as pl


def kernel(x, edge_index, edge_attr, batch, nn1_w1, nn1_b1, nn1_w2, nn1_b2, conv1_root, conv1_bias, nn2_w1, nn2_b1, nn2_w2, nn2_b2, conv2_root, conv2_bias, lin1_w, lin1_b, lin2_w, lin2_b):
    raise NotImplementedError("write your pallas kernel here")


# speedup vs baseline: 1.3719x; 1.3719x over previous
import functools

import jax
import jax.numpy as jnp
from jax import lax
from jax.experimental import pallas as pl
from jax.experimental.pallas import tpu as pltpu
from jax.experimental.pallas import tpu_sc as plsc

E = 160000
N = 10000
G = 64
CHUNK = 128
NUM_CHUNKS = E // CHUNK
NC = 2
NS = 16
NW = NC * NS
CPW = (NUM_CHUNKS + NW - 1) // NW
ROWS_PER_TILE = N // NS


def _sc_mesh():
    return plsc.VectorSubcoreMesh(core_axis_name="c", subcore_axis_name="s")


_SC_PARAMS = pltpu.CompilerParams(use_tc_tiling_on_sc=False)


def _sc_gather(table, idx, D):
    @functools.partial(
        pl.kernel,
        mesh=_sc_mesh(),
        out_type=jax.ShapeDtypeStruct((E, D), jnp.float32),
        scratch_types=[
            pltpu.VMEM((CHUNK,), jnp.int32),
            pltpu.VMEM((CHUNK, D), jnp.float32),
            pltpu.SemaphoreType.DMA,
        ],
        compiler_params=_SC_PARAMS,
    )
    def k(table_hbm, idx_hbm, out_hbm, idx_v, rows_v, sem):
        cid = lax.axis_index("c")
        sid = lax.axis_index("s")
        wid = sid * NC + cid

        def body(j, carry):
            c = wid + NW * j

            @pl.when(c < NUM_CHUNKS)
            def _():
                pltpu.sync_copy(idx_hbm.at[pl.ds(c * CHUNK, CHUNK)], idx_v)
                pltpu.async_copy(table_hbm.at[idx_v], rows_v, sem).wait()
                pltpu.sync_copy(rows_v, out_hbm.at[pl.ds(c * CHUNK, CHUNK)])

            return carry

        lax.fori_loop(0, CPW, body, 0)

    return k(table, idx)


def _sc_scatter_add(msg, dst, D):
    @functools.partial(
        pl.kernel,
        mesh=_sc_mesh(),
        out_type=jax.ShapeDtypeStruct((NC, N, D), jnp.float32),
        scratch_types=[
            pltpu.VMEM((CHUNK,), jnp.int32),
            pltpu.VMEM((CHUNK, D), jnp.float32),
            pltpu.VMEM_SHARED((N, D), jnp.float32),
            pltpu.SemaphoreType.DMA,
        ],
        compiler_params=_SC_PARAMS,
    )
    def k(msg_hbm, dst_hbm, out_hbm, idx_v, rows_v, acc_shared, sem):
        cid = lax.axis_index("c")
        sid = lax.axis_index("s")
        wid = sid * NC + cid

        def zrow(r, carry):
            for c4 in range(D // 16):
                rows_v[r, pl.ds(c4 * 16, 16)] = jnp.zeros((16,), jnp.float32)
            return carry

        lax.fori_loop(0, CHUNK, zrow, 0)
        base = sid * ROWS_PER_TILE
        off = 0
        for blk in (128, 128, 128, 128, 113):
            pltpu.sync_copy(rows_v.at[pl.ds(0, blk)],
                            acc_shared.at[pl.ds(base + off, blk)])
            off += blk
        plsc.subcore_barrier()

        def body(j, carry):
            c = wid + NW * j

            @pl.when(c < NUM_CHUNKS)
            def _():
                pltpu.sync_copy(dst_hbm.at[pl.ds(c * CHUNK, CHUNK)], idx_v)
                pltpu.sync_copy(msg_hbm.at[pl.ds(c * CHUNK, CHUNK)], rows_v)
                pltpu.sync_copy(rows_v, acc_shared.at[idx_v], add=True)

            return carry

        lax.fori_loop(0, CPW, body, 0)
        plsc.subcore_barrier()
        pltpu.sync_copy(acc_shared.at[pl.ds(base, ROWS_PER_TILE)],
                        out_hbm.at[cid, pl.ds(base, ROWS_PER_TILE)])

    return k(msg, dst)


_T1 = 640


def _tc_msg1(ea, xj1, w1, b1, w2, b2):
    def body(ea_ref, xj_ref, w1_ref, b1_ref, w2_ref, b2_ref, out_ref):
        ea = ea_ref[...]
        h = jnp.maximum(
            ea[:, 0:1] * w1_ref[0:1, :] + ea[:, 1:2] * w1_ref[1:2, :]
            + b1_ref[...], 0.0)
        ew = jnp.dot(h, w2_ref[...], preferred_element_type=jnp.float32) \
            + b2_ref[...]
        out_ref[...] = xj_ref[...] * ew

    return pl.pallas_call(
        body,
        grid=(E // _T1,),
        in_specs=[
            pl.BlockSpec((_T1, 2), lambda i: (i, 0)),
            pl.BlockSpec((_T1, 1), lambda i: (i, 0)),
            pl.BlockSpec((2, 32), lambda i: (0, 0)),
            pl.BlockSpec((1, 32), lambda i: (0, 0)),
            pl.BlockSpec((32, 32), lambda i: (0, 0)),
            pl.BlockSpec((1, 32), lambda i: (0, 0)),
        ],
        out_specs=pl.BlockSpec((_T1, 32), lambda i: (i, 0)),
        out_shape=jax.ShapeDtypeStruct((E, 32), jnp.float32),
    )(ea, xj1, w1, b1.reshape(1, 32), w2, b2.reshape(1, 32))


def _tc_msg2(ea, xj2, w1, b1, w2, b2):
    def body(ea_ref, xj_ref, w1_ref, b1_ref, w2_ref, b2_ref, out_ref):
        ea = ea_ref[...]
        h = jnp.maximum(
            ea[:, 0:1] * w1_ref[0:1, :] + ea[:, 1:2] * w1_ref[1:2, :]
            + b1_ref[...], 0.0)
        ew = jnp.dot(h, w2_ref[...], preferred_element_type=jnp.float32) \
            + b2_ref[...]
        xj = xj_ref[...]
        acc = xj[:, 0:1] * ew[:, 0:64]
        for i in range(1, 32):
            acc = acc + xj[:, i:i + 1] * ew[:, i * 64:(i + 1) * 64]
        out_ref[...] = acc

    return pl.pallas_call(
        body,
        grid=(E // _T1,),
        in_specs=[
            pl.BlockSpec((_T1, 2), lambda i: (i, 0)),
            pl.BlockSpec((_T1, 32), lambda i: (i, 0)),
            pl.BlockSpec((2, 64), lambda i: (0, 0)),
            pl.BlockSpec((1, 64), lambda i: (0, 0)),
            pl.BlockSpec((64, 2048), lambda i: (0, 0)),
            pl.BlockSpec((1, 2048), lambda i: (0, 0)),
        ],
        out_specs=pl.BlockSpec((_T1, 64), lambda i: (i, 0)),
        out_shape=jax.ShapeDtypeStruct((E, 64), jnp.float32),
    )(ea, xj2, w1, b1.reshape(1, 64), w2, b2.reshape(1, 2048))


def _elu(a):
    return jnp.where(a > 0, a, jnp.exp(jnp.minimum(a, 0.0)) - 1.0)


def _tc_h1(p, x, root, bias):
    def body(p_ref, x_ref, root_ref, b_ref, out_ref):
        a = p_ref[0] + p_ref[1] + x_ref[...] * root_ref[...] + b_ref[...]
        out_ref[...] = _elu(a)

    return pl.pallas_call(
        body,
        out_shape=jax.ShapeDtypeStruct((N, 32), jnp.float32),
    )(p, x, root, bias.reshape(1, 32))


def _tc_tail(p2, h1, root2, bias2, batch_row,
             lin1_w, lin1_b, lin2_w, lin2_b):
    def body(p_ref, h1_ref, root_ref, b_ref, batch_ref,
             w1_ref, b1_ref, w2_ref, b2_ref, out_ref):
        a = p_ref[0] + p_ref[1] \
            + jnp.dot(h1_ref[...], root_ref[...],
                      preferred_element_type=jnp.float32) + b_ref[...]
        h2 = _elu(a)
        gids = lax.broadcasted_iota(jnp.int32, (G, N), 0)
        oh = jnp.where(gids == batch_ref[...], 1.0, 0.0)
        s = jnp.dot(oh, h2, preferred_element_type=jnp.float32)
        cnt = jnp.sum(oh, axis=1, keepdims=True)
        pooled = s / jnp.maximum(cnt, 1.0)
        z = _elu(jnp.dot(pooled, w1_ref[...],
                         preferred_element_type=jnp.float32) + b1_ref[...])
        logits = jnp.dot(z, w2_ref[...],
                         preferred_element_type=jnp.float32) + b2_ref[...]
        m = jnp.max(logits, axis=1, keepdims=True)
        lse = jnp.log(jnp.sum(jnp.exp(logits - m), axis=1, keepdims=True)) + m
        out_ref[...] = logits - lse

    return pl.pallas_call(
        body,
        out_shape=jax.ShapeDtypeStruct((G, 10), jnp.float32),
    )(p2, h1, root2, bias2.reshape(1, 64), batch_row,
      lin1_w, lin1_b.reshape(1, 128), lin2_w, lin2_b.reshape(1, 10))


def kernel(x, edge_index, edge_attr, batch,
           nn1_w1, nn1_b1, nn1_w2, nn1_b2, conv1_root, conv1_bias,
           nn2_w1, nn2_b1, nn2_w2, nn2_b2, conv2_root, conv2_bias,
           lin1_w, lin1_b, lin2_w, lin2_b):
    src = edge_index[0]
    dst = edge_index[1]

    xj1 = _sc_gather(x, src, 1)
    msg1 = _tc_msg1(edge_attr, xj1, nn1_w1, nn1_b1, nn1_w2, nn1_b2)
    p1 = _sc_scatter_add(msg1, dst, 32)
    h1 = _tc_h1(p1, x, conv1_root, conv1_bias)
    xj2 = _sc_gather(h1, src, 32)
    msg2 = _tc_msg2(edge_attr, xj2, nn2_w1, nn2_b1, nn2_w2, nn2_b2)
    p2 = _sc_scatter_add(msg2, dst, 64)
    return _tc_tail(p2, h1, conv2_root, conv2_bias,
                    batch.reshape(1, N).astype(jnp.int32),
                    lin1_w, lin1_b, lin2_w, lin2_b)


# speedup vs baseline: 1.8672x; 1.3610x over previous
import functools

import jax
import jax.numpy as jnp
from jax import lax
from jax.experimental import pallas as pl
from jax.experimental.pallas import tpu as pltpu
from jax.experimental.pallas import tpu_sc as plsc

E = 160000
N = 10000
G = 64
CHUNK = 128
NUM_CHUNKS = E // CHUNK
NC = 2
NS = 16
NW = NC * NS
CPW = (NUM_CHUNKS + NW - 1) // NW
ROWS_PER_TILE = N // NS


def _sc_mesh():
    return plsc.VectorSubcoreMesh(core_axis_name="c", subcore_axis_name="s")


_SC_PARAMS = pltpu.CompilerParams(use_tc_tiling_on_sc=False)


def _sc_gather(table, idx, D):
    @functools.partial(
        pl.kernel,
        mesh=_sc_mesh(),
        out_type=jax.ShapeDtypeStruct((E, D), jnp.float32),
        scratch_types=[
            pltpu.VMEM((CHUNK,), jnp.int32),
            pltpu.VMEM((CHUNK, D), jnp.float32),
            pltpu.SemaphoreType.DMA,
        ],
        compiler_params=_SC_PARAMS,
    )
    def k(table_hbm, idx_hbm, out_hbm, idx_v, rows_v, sem):
        cid = lax.axis_index("c")
        sid = lax.axis_index("s")
        wid = sid * NC + cid

        def body(j, carry):
            c = wid + NW * j

            @pl.when(c < NUM_CHUNKS)
            def _():
                pltpu.sync_copy(idx_hbm.at[pl.ds(c * CHUNK, CHUNK)], idx_v)
                pltpu.async_copy(table_hbm.at[idx_v], rows_v, sem).wait()
                pltpu.sync_copy(rows_v, out_hbm.at[pl.ds(c * CHUNK, CHUNK)])

            return carry

        lax.fori_loop(0, CPW, body, 0)

    return k(table, idx)


def _sc_scatter_add(msg, dst, D):
    @functools.partial(
        pl.kernel,
        mesh=_sc_mesh(),
        out_type=jax.ShapeDtypeStruct((NC, N, D), jnp.float32),
        scratch_types=[
            pltpu.VMEM((CHUNK,), jnp.int32),
            pltpu.VMEM((CHUNK, D), jnp.float32),
            pltpu.VMEM_SHARED((N, D), jnp.float32),
            pltpu.SemaphoreType.DMA,
        ],
        compiler_params=_SC_PARAMS,
    )
    def k(msg_hbm, dst_hbm, out_hbm, idx_v, rows_v, acc_shared, sem):
        cid = lax.axis_index("c")
        sid = lax.axis_index("s")
        wid = sid * NC + cid

        def zrow(r, carry):
            for c4 in range(D // 16):
                rows_v[r, pl.ds(c4 * 16, 16)] = jnp.zeros((16,), jnp.float32)
            return carry

        lax.fori_loop(0, CHUNK, zrow, 0)
        base = sid * ROWS_PER_TILE
        off = 0
        for blk in (128, 128, 128, 128, 113):
            pltpu.sync_copy(rows_v.at[pl.ds(0, blk)],
                            acc_shared.at[pl.ds(base + off, blk)])
            off += blk
        plsc.subcore_barrier()

        def body(j, carry):
            c = wid + NW * j

            @pl.when(c < NUM_CHUNKS)
            def _():
                pltpu.sync_copy(dst_hbm.at[pl.ds(c * CHUNK, CHUNK)], idx_v)
                pltpu.sync_copy(msg_hbm.at[pl.ds(c * CHUNK, CHUNK)], rows_v)
                pltpu.sync_copy(rows_v, acc_shared.at[idx_v], add=True)

            return carry

        lax.fori_loop(0, CPW, body, 0)
        plsc.subcore_barrier()
        pltpu.sync_copy(acc_shared.at[pl.ds(base, ROWS_PER_TILE)],
                        out_hbm.at[cid, pl.ds(base, ROWS_PER_TILE)])

    return k(msg, dst)


_T1 = 640


def _tc_msg1(ea, xj1, w1, b1, w2, b2):
    def body(ea_ref, xj_ref, w1_ref, b1_ref, w2_ref, b2_ref, out_ref):
        ea = ea_ref[...]
        h = jnp.maximum(
            ea[:, 0:1] * w1_ref[0:1, :] + ea[:, 1:2] * w1_ref[1:2, :]
            + b1_ref[...], 0.0)
        ew = jnp.dot(h, w2_ref[...], preferred_element_type=jnp.float32) \
            + b2_ref[...]
        out_ref[...] = xj_ref[...] * ew

    return pl.pallas_call(
        body,
        grid=(E // _T1,),
        in_specs=[
            pl.BlockSpec((_T1, 2), lambda i: (i, 0)),
            pl.BlockSpec((_T1, 1), lambda i: (i, 0)),
            pl.BlockSpec((2, 32), lambda i: (0, 0)),
            pl.BlockSpec((1, 32), lambda i: (0, 0)),
            pl.BlockSpec((32, 32), lambda i: (0, 0)),
            pl.BlockSpec((1, 32), lambda i: (0, 0)),
        ],
        out_specs=pl.BlockSpec((_T1, 32), lambda i: (i, 0)),
        out_shape=jax.ShapeDtypeStruct((E, 32), jnp.float32),
    )(ea, xj1, w1, b1.reshape(1, 32), w2, b2.reshape(1, 32))


def _tc_msg2(ea, xj2, w1, b1, w2, b2):
    col = jnp.arange(2048, dtype=jnp.int32)
    expand = (col[None, :] // 64
              == jnp.arange(32, dtype=jnp.int32)[:, None]).astype(jnp.bfloat16)
    reduce = (col[:, None] % 64
              == jnp.arange(64, dtype=jnp.int32)[None, :]).astype(jnp.bfloat16)
    w2b = w2.astype(jnp.bfloat16)
    b2m = b2.reshape(32, 64)

    def body(ea_ref, xj_ref, w1_ref, b1_ref, w2_ref, b2_ref, r_ref, s_ref,
             out_ref):
        ea = ea_ref[...]
        h = jnp.maximum(
            ea[:, 0:1] * w1_ref[0:1, :] + ea[:, 1:2] * w1_ref[1:2, :]
            + b1_ref[...], 0.0)
        ew = jnp.dot(h.astype(jnp.bfloat16), w2_ref[...],
                     preferred_element_type=jnp.float32)
        xjb = xj_ref[...].astype(jnp.bfloat16)
        xrep = jnp.dot(xjb, r_ref[...],
                       preferred_element_type=jnp.float32)
        out_ref[...] = (
            jnp.dot((xrep * ew).astype(jnp.bfloat16), s_ref[...],
                    preferred_element_type=jnp.float32)
            + jnp.dot(xj_ref[...], b2_ref[...],
                      preferred_element_type=jnp.float32))

    return pl.pallas_call(
        body,
        grid=(E // _T1,),
        in_specs=[
            pl.BlockSpec((_T1, 2), lambda i: (i, 0)),
            pl.BlockSpec((_T1, 32), lambda i: (i, 0)),
            pl.BlockSpec((2, 64), lambda i: (0, 0)),
            pl.BlockSpec((1, 64), lambda i: (0, 0)),
            pl.BlockSpec((64, 2048), lambda i: (0, 0)),
            pl.BlockSpec((32, 64), lambda i: (0, 0)),
            pl.BlockSpec((32, 2048), lambda i: (0, 0)),
            pl.BlockSpec((2048, 64), lambda i: (0, 0)),
        ],
        out_specs=pl.BlockSpec((_T1, 64), lambda i: (i, 0)),
        out_shape=jax.ShapeDtypeStruct((E, 64), jnp.float32),
    )(ea, xj2, w1, b1.reshape(1, 64), w2b, b2m, expand, reduce)


def _elu(a):
    return jnp.where(a > 0, a, jnp.exp(jnp.minimum(a, 0.0)) - 1.0)


def _tc_h1(p, x, root, bias):
    def body(p_ref, x_ref, root_ref, b_ref, out_ref):
        a = p_ref[0] + p_ref[1] + x_ref[...] * root_ref[...] + b_ref[...]
        out_ref[...] = _elu(a)

    return pl.pallas_call(
        body,
        out_shape=jax.ShapeDtypeStruct((N, 32), jnp.float32),
    )(p, x, root, bias.reshape(1, 32))


def _tc_tail(p2, h1, root2, bias2, batch_row,
             lin1_w, lin1_b, lin2_w, lin2_b):
    def body(p_ref, h1_ref, root_ref, b_ref, batch_ref,
             w1_ref, b1_ref, w2_ref, b2_ref, out_ref):
        a = p_ref[0] + p_ref[1] \
            + jnp.dot(h1_ref[...], root_ref[...],
                      preferred_element_type=jnp.float32) + b_ref[...]
        h2 = _elu(a)
        gids = lax.broadcasted_iota(jnp.int32, (G, N), 0)
        oh = jnp.where(gids == batch_ref[...], 1.0, 0.0)
        s = jnp.dot(oh, h2, preferred_element_type=jnp.float32)
        cnt = jnp.sum(oh, axis=1, keepdims=True)
        pooled = s / jnp.maximum(cnt, 1.0)
        z = _elu(jnp.dot(pooled, w1_ref[...],
                         preferred_element_type=jnp.float32) + b1_ref[...])
        logits = jnp.dot(z, w2_ref[...],
                         preferred_element_type=jnp.float32) + b2_ref[...]
        m = jnp.max(logits, axis=1, keepdims=True)
        lse = jnp.log(jnp.sum(jnp.exp(logits - m), axis=1, keepdims=True)) + m
        out_ref[...] = logits - lse

    return pl.pallas_call(
        body,
        out_shape=jax.ShapeDtypeStruct((G, 10), jnp.float32),
    )(p2, h1, root2, bias2.reshape(1, 64), batch_row,
      lin1_w, lin1_b.reshape(1, 128), lin2_w, lin2_b.reshape(1, 10))


def kernel(x, edge_index, edge_attr, batch,
           nn1_w1, nn1_b1, nn1_w2, nn1_b2, conv1_root, conv1_bias,
           nn2_w1, nn2_b1, nn2_w2, nn2_b2, conv2_root, conv2_bias,
           lin1_w, lin1_b, lin2_w, lin2_b):
    src = edge_index[0]
    dst = edge_index[1]

    xj1 = _sc_gather(x, src, 1)
    msg1 = _tc_msg1(edge_attr, xj1, nn1_w1, nn1_b1, nn1_w2, nn1_b2)
    p1 = _sc_scatter_add(msg1, dst, 32)
    h1 = _tc_h1(p1, x, conv1_root, conv1_bias)
    xj2 = _sc_gather(h1, src, 32)
    msg2 = _tc_msg2(edge_attr, xj2, nn2_w1, nn2_b1, nn2_w2, nn2_b2)
    p2 = _sc_scatter_add(msg2, dst, 64)
    return _tc_tail(p2, h1, conv2_root, conv2_bias,
                    batch.reshape(1, N).astype(jnp.int32),
                    lin1_w, lin1_b, lin2_w, lin2_b)


# speedup vs baseline: 3.0616x; 1.6397x over previous
import functools

import jax
import jax.numpy as jnp
from jax import lax
from jax.experimental import pallas as pl
from jax.experimental.pallas import tpu as pltpu
from jax.experimental.pallas import tpu_sc as plsc

E = 160000
N = 10000
G = 64
CHUNK = 128
NUM_CHUNKS = E // CHUNK
NC = 2
NS = 16
NW = NC * NS
CPW = (NUM_CHUNKS + NW - 1) // NW
ROWS_PER_TILE = N // NS


def _sc_mesh():
    return plsc.VectorSubcoreMesh(core_axis_name="c", subcore_axis_name="s")


_SC_PARAMS = pltpu.CompilerParams(use_tc_tiling_on_sc=False)


def _sc_gather(table, idx, D):
    @functools.partial(
        pl.kernel,
        mesh=_sc_mesh(),
        out_type=jax.ShapeDtypeStruct((E, D), jnp.float32),
        scratch_types=[
            pltpu.VMEM((CHUNK,), jnp.int32),
            pltpu.VMEM((CHUNK, D), jnp.float32),
            pltpu.SemaphoreType.DMA,
        ],
        compiler_params=_SC_PARAMS,
    )
    def k(table_hbm, idx_hbm, out_hbm, idx_v, rows_v, sem):
        cid = lax.axis_index("c")
        sid = lax.axis_index("s")
        wid = sid * NC + cid

        def body(j, carry):
            c = wid + NW * j

            @pl.when(c < NUM_CHUNKS)
            def _():
                pltpu.sync_copy(idx_hbm.at[pl.ds(c * CHUNK, CHUNK)], idx_v)
                pltpu.async_copy(table_hbm.at[idx_v], rows_v, sem).wait()
                pltpu.sync_copy(rows_v, out_hbm.at[pl.ds(c * CHUNK, CHUNK)])

            return carry

        lax.fori_loop(0, CPW, body, 0)

    return k(table, idx)


def _sc_scatter_add(msg, dst, D):
    @functools.partial(
        pl.kernel,
        mesh=_sc_mesh(),
        out_type=jax.ShapeDtypeStruct((NC, N, D), jnp.float32),
        scratch_types=[
            pltpu.VMEM((CHUNK,), jnp.int32),
            pltpu.VMEM((CHUNK, D), jnp.float32),
            pltpu.VMEM_SHARED((N, D), jnp.float32),
            pltpu.SemaphoreType.DMA,
        ],
        compiler_params=_SC_PARAMS,
    )
    def k(msg_hbm, dst_hbm, out_hbm, idx_v, rows_v, acc_shared, sem):
        cid = lax.axis_index("c")
        sid = lax.axis_index("s")
        wid = sid * NC + cid

        def zrow(r, carry):
            for c4 in range(D // 16):
                rows_v[r, pl.ds(c4 * 16, 16)] = jnp.zeros((16,), jnp.float32)
            return carry

        lax.fori_loop(0, CHUNK, zrow, 0)
        base = sid * ROWS_PER_TILE
        off = 0
        for blk in (128, 128, 128, 128, 113):
            pltpu.sync_copy(rows_v.at[pl.ds(0, blk)],
                            acc_shared.at[pl.ds(base + off, blk)])
            off += blk
        plsc.subcore_barrier()

        def body(j, carry):
            c = wid + NW * j

            @pl.when(c < NUM_CHUNKS)
            def _():
                pltpu.sync_copy(dst_hbm.at[pl.ds(c * CHUNK, CHUNK)], idx_v)
                pltpu.sync_copy(msg_hbm.at[pl.ds(c * CHUNK, CHUNK)], rows_v)
                pltpu.sync_copy(rows_v, acc_shared.at[idx_v], add=True)

            return carry

        lax.fori_loop(0, CPW, body, 0)
        plsc.subcore_barrier()
        pltpu.sync_copy(acc_shared.at[pl.ds(base, ROWS_PER_TILE)],
                        out_hbm.at[cid, pl.ds(base, ROWS_PER_TILE)])

    return k(msg, dst)


_T1 = 3200
_T2 = 640


def _tc_msg1(ea, xj1, w1, b1, w2, b2):
    def body(ea_ref, xj_ref, w1_ref, b1_ref, w2_ref, b2_ref, out_ref):
        ea = ea_ref[...]
        h = jnp.maximum(
            ea[:, 0:1] * w1_ref[0:1, :] + ea[:, 1:2] * w1_ref[1:2, :]
            + b1_ref[...], 0.0)
        ew = jnp.dot(h, w2_ref[...], preferred_element_type=jnp.float32) \
            + b2_ref[...]
        out_ref[...] = xj_ref[...] * ew

    return pl.pallas_call(
        body,
        grid=(E // _T1,),
        in_specs=[
            pl.BlockSpec((_T1, 2), lambda i: (i, 0)),
            pl.BlockSpec((_T1, 1), lambda i: (i, 0)),
            pl.BlockSpec((2, 32), lambda i: (0, 0)),
            pl.BlockSpec((1, 32), lambda i: (0, 0)),
            pl.BlockSpec((32, 32), lambda i: (0, 0)),
            pl.BlockSpec((1, 32), lambda i: (0, 0)),
        ],
        out_specs=pl.BlockSpec((_T1, 32), lambda i: (i, 0)),
        out_shape=jax.ShapeDtypeStruct((E, 32), jnp.float32),
    )(ea, xj1, w1, b1.reshape(1, 32), w2, b2.reshape(1, 32))


def _tc_msg2(ea, xj2, w1, b1, w2, b2):
    s2 = w2.reshape(64, 32, 64).transpose(2, 1, 0).reshape(64, 2048)
    b2t = b2.reshape(32, 64).T
    w1t = w1.T
    ea_t = ea.T

    def body(ea_ref, xj_ref, w1_ref, b1_ref, s2_ref, b2_ref, out_ref):
        h_t = jnp.maximum(
            jnp.dot(w1_ref[...], ea_ref[...],
                    preferred_element_type=jnp.float32)
            + b1_ref[...], 0.0)
        xj_t = xj_ref[...].T
        xh = jnp.concatenate(
            [h_t * xj_t[i:i + 1, :] for i in range(32)],
            axis=0)
        msg_t = (
            jnp.dot(s2_ref[...], xh, preferred_element_type=jnp.float32)
            + jnp.dot(b2_ref[...], xj_t,
                      preferred_element_type=jnp.float32))
        out_ref[...] = msg_t.T

    return pl.pallas_call(
        body,
        grid=(E // _T2,),
        in_specs=[
            pl.BlockSpec((2, _T2), lambda i: (0, i)),
            pl.BlockSpec((_T2, 32), lambda i: (i, 0)),
            pl.BlockSpec((64, 2), lambda i: (0, 0)),
            pl.BlockSpec((64, 1), lambda i: (0, 0)),
            pl.BlockSpec((64, 2048), lambda i: (0, 0)),
            pl.BlockSpec((64, 32), lambda i: (0, 0)),
        ],
        out_specs=pl.BlockSpec((_T2, 64), lambda i: (i, 0)),
        out_shape=jax.ShapeDtypeStruct((E, 64), jnp.float32),
    )(ea_t, xj2, w1t, b1.reshape(64, 1), s2, b2t)


def _elu(a):
    return jnp.where(a > 0, a, jnp.exp(jnp.minimum(a, 0.0)) - 1.0)


def _tc_h1(p, x, root, bias):
    def body(p_ref, x_ref, root_ref, b_ref, out_ref):
        a = p_ref[0] + p_ref[1] + x_ref[...] * root_ref[...] + b_ref[...]
        out_ref[...] = _elu(a)

    return pl.pallas_call(
        body,
        out_shape=jax.ShapeDtypeStruct((N, 32), jnp.float32),
    )(p, x, root, bias.reshape(1, 32))


def _tc_tail(p2, h1, root2, bias2, batch_row,
             lin1_w, lin1_b, lin2_w, lin2_b):
    def body(p_ref, h1_ref, root_ref, b_ref, batch_ref,
             w1_ref, b1_ref, w2_ref, b2_ref, out_ref):
        a = p_ref[0] + p_ref[1] \
            + jnp.dot(h1_ref[...], root_ref[...],
                      preferred_element_type=jnp.float32) + b_ref[...]
        h2 = _elu(a)
        gids = lax.broadcasted_iota(jnp.int32, (G, N), 0)
        oh = jnp.where(gids == batch_ref[...], 1.0, 0.0)
        s = jnp.dot(oh, h2, preferred_element_type=jnp.float32)
        cnt = jnp.sum(oh, axis=1, keepdims=True)
        pooled = s / jnp.maximum(cnt, 1.0)
        z = _elu(jnp.dot(pooled, w1_ref[...],
                         preferred_element_type=jnp.float32) + b1_ref[...])
        logits = jnp.dot(z, w2_ref[...],
                         preferred_element_type=jnp.float32) + b2_ref[...]
        m = jnp.max(logits, axis=1, keepdims=True)
        lse = jnp.log(jnp.sum(jnp.exp(logits - m), axis=1, keepdims=True)) + m
        out_ref[...] = logits - lse

    return pl.pallas_call(
        body,
        out_shape=jax.ShapeDtypeStruct((G, 10), jnp.float32),
    )(p2, h1, root2, bias2.reshape(1, 64), batch_row,
      lin1_w, lin1_b.reshape(1, 128), lin2_w, lin2_b.reshape(1, 10))


def kernel(x, edge_index, edge_attr, batch,
           nn1_w1, nn1_b1, nn1_w2, nn1_b2, conv1_root, conv1_bias,
           nn2_w1, nn2_b1, nn2_w2, nn2_b2, conv2_root, conv2_bias,
           lin1_w, lin1_b, lin2_w, lin2_b):
    src = edge_index[0]
    dst = edge_index[1]

    xj1 = _sc_gather(x, src, 1)
    msg1 = _tc_msg1(edge_attr, xj1, nn1_w1, nn1_b1, nn1_w2, nn1_b2)
    p1 = _sc_scatter_add(msg1, dst, 32)
    h1 = _tc_h1(p1, x, conv1_root, conv1_bias)
    xj2 = _sc_gather(h1, src, 32)
    msg2 = _tc_msg2(edge_attr, xj2, nn2_w1, nn2_b1, nn2_w2, nn2_b2)
    p2 = _sc_scatter_add(msg2, dst, 64)
    return _tc_tail(p2, h1, conv2_root, conv2_bias,
                    batch.reshape(1, N).astype(jnp.int32),
                    lin1_w, lin1_b, lin2_w, lin2_b)


# speedup vs baseline: 3.1809x; 1.0390x over previous
import functools

import jax
import jax.numpy as jnp
from jax import lax
from jax.experimental import pallas as pl
from jax.experimental.pallas import tpu as pltpu
from jax.experimental.pallas import tpu_sc as plsc

E = 160000
N = 10000
G = 64
CHUNK = 128
NUM_CHUNKS = E // CHUNK
NC = 2
NS = 16
NW = NC * NS
CPW = (NUM_CHUNKS + NW - 1) // NW
ROWS_PER_TILE = N // NS


def _sc_mesh():
    return plsc.VectorSubcoreMesh(core_axis_name="c", subcore_axis_name="s")


_SC_PARAMS = pltpu.CompilerParams(use_tc_tiling_on_sc=False)


def _sc_gather(table, idx, D):
    @functools.partial(
        pl.kernel,
        mesh=_sc_mesh(),
        out_type=jax.ShapeDtypeStruct((E, D), jnp.float32),
        scratch_types=[
            pltpu.VMEM((CHUNK,), jnp.int32),
            pltpu.VMEM((CHUNK, D), jnp.float32),
            pltpu.SemaphoreType.DMA,
        ],
        compiler_params=_SC_PARAMS,
    )
    def k(table_hbm, idx_hbm, out_hbm, idx_v, rows_v, sem):
        cid = lax.axis_index("c")
        sid = lax.axis_index("s")
        wid = sid * NC + cid

        def body(j, carry):
            c = wid + NW * j

            @pl.when(c < NUM_CHUNKS)
            def _():
                pltpu.sync_copy(idx_hbm.at[pl.ds(c * CHUNK, CHUNK)], idx_v)
                pltpu.async_copy(table_hbm.at[idx_v], rows_v, sem).wait()
                pltpu.sync_copy(rows_v, out_hbm.at[pl.ds(c * CHUNK, CHUNK)])

            return carry

        lax.fori_loop(0, CPW, body, 0)

    return k(table, idx)


def _sc_scatter_add(msg, dst, D):
    @functools.partial(
        pl.kernel,
        mesh=_sc_mesh(),
        out_type=jax.ShapeDtypeStruct((NC, N, D), jnp.float32),
        scratch_types=[
            pltpu.VMEM((CHUNK,), jnp.int32),
            pltpu.VMEM((CHUNK, D), jnp.float32),
            pltpu.VMEM_SHARED((N, D), jnp.float32),
            pltpu.SemaphoreType.DMA,
        ],
        compiler_params=_SC_PARAMS,
    )
    def k(msg_hbm, dst_hbm, out_hbm, idx_v, rows_v, acc_shared, sem):
        cid = lax.axis_index("c")
        sid = lax.axis_index("s")
        wid = sid * NC + cid

        def zrow(r, carry):
            for c4 in range(D // 16):
                rows_v[r, pl.ds(c4 * 16, 16)] = jnp.zeros((16,), jnp.float32)
            return carry

        lax.fori_loop(0, CHUNK, zrow, 0)
        base = sid * ROWS_PER_TILE
        off = 0
        for blk in (128, 128, 128, 128, 113):
            pltpu.sync_copy(rows_v.at[pl.ds(0, blk)],
                            acc_shared.at[pl.ds(base + off, blk)])
            off += blk
        plsc.subcore_barrier()

        def body(j, carry):
            c = wid + NW * j

            @pl.when(c < NUM_CHUNKS)
            def _():
                pltpu.sync_copy(dst_hbm.at[pl.ds(c * CHUNK, CHUNK)], idx_v)
                pltpu.sync_copy(msg_hbm.at[pl.ds(c * CHUNK, CHUNK)], rows_v)
                pltpu.sync_copy(rows_v, acc_shared.at[idx_v], add=True)

            return carry

        lax.fori_loop(0, CPW, body, 0)
        plsc.subcore_barrier()
        pltpu.sync_copy(acc_shared.at[pl.ds(base, ROWS_PER_TILE)],
                        out_hbm.at[cid, pl.ds(base, ROWS_PER_TILE)])

    return k(msg, dst)


_T1 = 3200
_NB1 = N // (E // _T1)
_T2 = 640


def _tc_ew1_x32(ea, x, w1, b1, w2, b2):
    def body(ea_ref, x_ref, w1_ref, b1_ref, w2_ref, b2_ref,
             ew_ref, x32_ref):
        ea = ea_ref[...]
        h = jnp.maximum(
            ea[:, 0:1] * w1_ref[0:1, :] + ea[:, 1:2] * w1_ref[1:2, :]
            + b1_ref[...], 0.0)
        ew_ref[...] = jnp.dot(h, w2_ref[...],
                              preferred_element_type=jnp.float32) \
            + b2_ref[...]
        x32_ref[...] = jnp.broadcast_to(x_ref[...], (_NB1, 32))

    return pl.pallas_call(
        body,
        grid=(E // _T1,),
        in_specs=[
            pl.BlockSpec((_T1, 2), lambda i: (i, 0)),
            pl.BlockSpec((_NB1, 1), lambda i: (i, 0)),
            pl.BlockSpec((2, 32), lambda i: (0, 0)),
            pl.BlockSpec((1, 32), lambda i: (0, 0)),
            pl.BlockSpec((32, 32), lambda i: (0, 0)),
            pl.BlockSpec((1, 32), lambda i: (0, 0)),
        ],
        out_specs=[
            pl.BlockSpec((_T1, 32), lambda i: (i, 0)),
            pl.BlockSpec((_NB1, 32), lambda i: (i, 0)),
        ],
        out_shape=[
            jax.ShapeDtypeStruct((E, 32), jnp.float32),
            jax.ShapeDtypeStruct((N, 32), jnp.float32),
        ],
    )(ea, x, w1, b1.reshape(1, 32), w2, b2.reshape(1, 32))


def _sc_gather_mul_scatter(x32, ew1, src, dst):
    D = 32

    @functools.partial(
        pl.kernel,
        mesh=_sc_mesh(),
        out_type=jax.ShapeDtypeStruct((NC, N, D), jnp.float32),
        scratch_types=[
            pltpu.VMEM((CHUNK,), jnp.int32),
            pltpu.VMEM((CHUNK,), jnp.int32),
            pltpu.VMEM((CHUNK, D), jnp.float32),
            pltpu.VMEM((CHUNK, D), jnp.float32),
            pltpu.VMEM_SHARED((N, D), jnp.float32),
            pltpu.SemaphoreType.DMA,
        ],
        compiler_params=_SC_PARAMS,
    )
    def k(x32_hbm, ew_hbm, src_hbm, dst_hbm, out_hbm,
          src_v, dst_v, xrows_v, mrows_v, acc_shared, sem):
        cid = lax.axis_index("c")
        sid = lax.axis_index("s")
        wid = sid * NC + cid

        def zrow(r, carry):
            for c4 in range(D // 16):
                mrows_v[r, pl.ds(c4 * 16, 16)] = jnp.zeros((16,), jnp.float32)
            return carry

        lax.fori_loop(0, CHUNK, zrow, 0)
        base = sid * ROWS_PER_TILE
        off = 0
        for blk in (128, 128, 128, 128, 113):
            pltpu.sync_copy(mrows_v.at[pl.ds(0, blk)],
                            acc_shared.at[pl.ds(base + off, blk)])
            off += blk
        plsc.subcore_barrier()

        def body(j, carry):
            c = wid + NW * j

            @pl.when(c < NUM_CHUNKS)
            def _():
                pltpu.sync_copy(src_hbm.at[pl.ds(c * CHUNK, CHUNK)], src_v)
                pltpu.sync_copy(dst_hbm.at[pl.ds(c * CHUNK, CHUNK)], dst_v)
                pltpu.async_copy(x32_hbm.at[src_v], xrows_v, sem).wait()
                pltpu.sync_copy(ew_hbm.at[pl.ds(c * CHUNK, CHUNK)], mrows_v)

                def mulrow(r, carry2):
                    for c4 in range(D // 16):
                        sl = pl.ds(c4 * 16, 16)
                        mrows_v[r, sl] = mrows_v[r, sl] * xrows_v[r, sl]
                    return carry2

                lax.fori_loop(0, CHUNK, mulrow, 0)
                pltpu.sync_copy(mrows_v, acc_shared.at[dst_v], add=True)

            return carry

        lax.fori_loop(0, CPW, body, 0)
        plsc.subcore_barrier()
        pltpu.sync_copy(acc_shared.at[pl.ds(base, ROWS_PER_TILE)],
                        out_hbm.at[cid, pl.ds(base, ROWS_PER_TILE)])

    return k(x32, ew1, src, dst)


def _tc_msg2(ea, xj2, w1, b1, w2, b2):
    s2 = w2.reshape(64, 32, 64).transpose(2, 1, 0).reshape(64, 2048)
    b2t = b2.reshape(32, 64).T
    w1t = w1.T
    ea_t = ea.T

    def body(ea_ref, xj_ref, w1_ref, b1_ref, s2_ref, b2_ref, out_ref):
        h_t = jnp.maximum(
            jnp.dot(w1_ref[...], ea_ref[...],
                    preferred_element_type=jnp.float32)
            + b1_ref[...], 0.0)
        xj_t = xj_ref[...].T
        xh = jnp.concatenate(
            [h_t * xj_t[i:i + 1, :] for i in range(32)],
            axis=0)
        msg_t = (
            jnp.dot(s2_ref[...], xh, preferred_element_type=jnp.float32)
            + jnp.dot(b2_ref[...], xj_t,
                      preferred_element_type=jnp.float32))
        out_ref[...] = msg_t.T

    return pl.pallas_call(
        body,
        grid=(E // _T2,),
        in_specs=[
            pl.BlockSpec((2, _T2), lambda i: (0, i)),
            pl.BlockSpec((_T2, 32), lambda i: (i, 0)),
            pl.BlockSpec((64, 2), lambda i: (0, 0)),
            pl.BlockSpec((64, 1), lambda i: (0, 0)),
            pl.BlockSpec((64, 2048), lambda i: (0, 0)),
            pl.BlockSpec((64, 32), lambda i: (0, 0)),
        ],
        out_specs=pl.BlockSpec((_T2, 64), lambda i: (i, 0)),
        out_shape=jax.ShapeDtypeStruct((E, 64), jnp.float32),
    )(ea_t, xj2, w1t, b1.reshape(64, 1), s2, b2t)


def _elu(a):
    return jnp.where(a > 0, a, jnp.exp(jnp.minimum(a, 0.0)) - 1.0)


def _tc_h1(p, x, root, bias):
    def body(p_ref, x_ref, root_ref, b_ref, out_ref):
        a = p_ref[0] + p_ref[1] + x_ref[...] * root_ref[...] + b_ref[...]
        out_ref[...] = _elu(a)

    return pl.pallas_call(
        body,
        out_shape=jax.ShapeDtypeStruct((N, 32), jnp.float32),
    )(p, x, root, bias.reshape(1, 32))


def _tc_tail(p2, h1, root2, bias2, batch_row,
             lin1_w, lin1_b, lin2_w, lin2_b):
    def body(p_ref, h1_ref, root_ref, b_ref, batch_ref,
             w1_ref, b1_ref, w2_ref, b2_ref, out_ref):
        a = p_ref[0] + p_ref[1] \
            + jnp.dot(h1_ref[...], root_ref[...],
                      preferred_element_type=jnp.float32) + b_ref[...]
        h2 = _elu(a)
        gids = lax.broadcasted_iota(jnp.int32, (G, N), 0)
        oh = jnp.where(gids == batch_ref[...], 1.0, 0.0)
        s = jnp.dot(oh, h2, preferred_element_type=jnp.float32)
        cnt = jnp.sum(oh, axis=1, keepdims=True)
        pooled = s / jnp.maximum(cnt, 1.0)
        z = _elu(jnp.dot(pooled, w1_ref[...],
                         preferred_element_type=jnp.float32) + b1_ref[...])
        logits = jnp.dot(z, w2_ref[...],
                         preferred_element_type=jnp.float32) + b2_ref[...]
        m = jnp.max(logits, axis=1, keepdims=True)
        lse = jnp.log(jnp.sum(jnp.exp(logits - m), axis=1, keepdims=True)) + m
        out_ref[...] = logits - lse

    return pl.pallas_call(
        body,
        out_shape=jax.ShapeDtypeStruct((G, 10), jnp.float32),
    )(p2, h1, root2, bias2.reshape(1, 64), batch_row,
      lin1_w, lin1_b.reshape(1, 128), lin2_w, lin2_b.reshape(1, 10))


def kernel(x, edge_index, edge_attr, batch,
           nn1_w1, nn1_b1, nn1_w2, nn1_b2, conv1_root, conv1_bias,
           nn2_w1, nn2_b1, nn2_w2, nn2_b2, conv2_root, conv2_bias,
           lin1_w, lin1_b, lin2_w, lin2_b):
    src = edge_index[0]
    dst = edge_index[1]

    ew1, x32 = _tc_ew1_x32(edge_attr, x, nn1_w1, nn1_b1, nn1_w2, nn1_b2)
    p1 = _sc_gather_mul_scatter(x32, ew1, src, dst)
    h1 = _tc_h1(p1, x, conv1_root, conv1_bias)
    xj2 = _sc_gather(h1, src, 32)
    msg2 = _tc_msg2(edge_attr, xj2, nn2_w1, nn2_b1, nn2_w2, nn2_b2)
    p2 = _sc_scatter_add(msg2, dst, 64)
    return _tc_tail(p2, h1, conv2_root, conv2_bias,
                    batch.reshape(1, N).astype(jnp.int32),
                    lin1_w, lin1_b, lin2_w, lin2_b)


# speedup vs baseline: 3.3667x; 1.0584x over previous
import functools

import jax
import jax.numpy as jnp
from jax import lax
from jax.experimental import pallas as pl
from jax.experimental.pallas import tpu as pltpu
from jax.experimental.pallas import tpu_sc as plsc

E = 160000
N = 10000
G = 64
CHUNK = 128
NUM_CHUNKS = E // CHUNK
NC = 2
NS = 16
NW = NC * NS
CPW = (NUM_CHUNKS + NW - 1) // NW
ROWS_PER_TILE = N // NS


def _sc_mesh():
    return plsc.VectorSubcoreMesh(core_axis_name="c", subcore_axis_name="s")


_SC_PARAMS = pltpu.CompilerParams(use_tc_tiling_on_sc=False)


def _sc_gather(table, idx, D):
    ne = idx.shape[0]
    num_chunks = ne // CHUNK
    cpw = (num_chunks + NW - 1) // NW

    @functools.partial(
        pl.kernel,
        mesh=_sc_mesh(),
        out_type=jax.ShapeDtypeStruct((ne, D), jnp.float32),
        scratch_types=[
            pltpu.VMEM((CHUNK,), jnp.int32),
            pltpu.VMEM((CHUNK, D), jnp.float32),
            pltpu.SemaphoreType.DMA,
        ],
        compiler_params=_SC_PARAMS,
    )
    def k(table_hbm, idx_hbm, out_hbm, idx_v, rows_v, sem):
        cid = lax.axis_index("c")
        sid = lax.axis_index("s")
        wid = sid * NC + cid

        def body(j, carry):
            c = wid + NW * j

            @pl.when(c < num_chunks)
            def _():
                pltpu.sync_copy(idx_hbm.at[pl.ds(c * CHUNK, CHUNK)], idx_v)
                pltpu.async_copy(table_hbm.at[idx_v], rows_v, sem).wait()
                pltpu.sync_copy(rows_v, out_hbm.at[pl.ds(c * CHUNK, CHUNK)])

            return carry

        lax.fori_loop(0, cpw, body, 0)

    return k(table, idx)


def _sc_scatter_add(msg, dst, D):
    ne = msg.shape[0]
    num_chunks = ne // CHUNK
    cpw = (num_chunks + NW - 1) // NW

    @functools.partial(
        pl.kernel,
        mesh=_sc_mesh(),
        out_type=jax.ShapeDtypeStruct((NC, N, D), jnp.float32),
        scratch_types=[
            pltpu.VMEM((CHUNK,), jnp.int32),
            pltpu.VMEM((CHUNK, D), jnp.float32),
            pltpu.VMEM_SHARED((N, D), jnp.float32),
            pltpu.SemaphoreType.DMA,
        ],
        compiler_params=_SC_PARAMS,
    )
    def k(msg_hbm, dst_hbm, out_hbm, idx_v, rows_v, acc_shared, sem):
        cid = lax.axis_index("c")
        sid = lax.axis_index("s")
        wid = sid * NC + cid

        def zrow(r, carry):
            for c4 in range(D // 16):
                rows_v[r, pl.ds(c4 * 16, 16)] = jnp.zeros((16,), jnp.float32)
            return carry

        lax.fori_loop(0, CHUNK, zrow, 0)
        base = sid * ROWS_PER_TILE
        off = 0
        for blk in (128, 128, 128, 128, 113):
            pltpu.sync_copy(rows_v.at[pl.ds(0, blk)],
                            acc_shared.at[pl.ds(base + off, blk)])
            off += blk
        plsc.subcore_barrier()

        def body(j, carry):
            c = wid + NW * j

            @pl.when(c < num_chunks)
            def _():
                pltpu.sync_copy(dst_hbm.at[pl.ds(c * CHUNK, CHUNK)], idx_v)
                pltpu.sync_copy(msg_hbm.at[pl.ds(c * CHUNK, CHUNK)], rows_v)
                pltpu.sync_copy(rows_v, acc_shared.at[idx_v], add=True)

            return carry

        lax.fori_loop(0, cpw, body, 0)
        plsc.subcore_barrier()
        pltpu.sync_copy(acc_shared.at[pl.ds(base, ROWS_PER_TILE)],
                        out_hbm.at[cid, pl.ds(base, ROWS_PER_TILE)])

    return k(msg, dst)


_T1 = 3200
_NB1 = N // (E // _T1)
_T2 = 640


def _tc_ew1_x32(ea, x, w1, b1, w2, b2):
    def body(ea_ref, x_ref, w1_ref, b1_ref, w2_ref, b2_ref,
             ew_ref, x32_ref):
        ea = ea_ref[...]
        h = jnp.maximum(
            ea[:, 0:1] * w1_ref[0:1, :] + ea[:, 1:2] * w1_ref[1:2, :]
            + b1_ref[...], 0.0)
        ew_ref[...] = jnp.dot(h, w2_ref[...],
                              preferred_element_type=jnp.float32) \
            + b2_ref[...]
        x32_ref[...] = jnp.broadcast_to(x_ref[...], (_NB1, 32))

    return pl.pallas_call(
        body,
        grid=(E // _T1,),
        in_specs=[
            pl.BlockSpec((_T1, 2), lambda i: (i, 0)),
            pl.BlockSpec((_NB1, 1), lambda i: (i, 0)),
            pl.BlockSpec((2, 32), lambda i: (0, 0)),
            pl.BlockSpec((1, 32), lambda i: (0, 0)),
            pl.BlockSpec((32, 32), lambda i: (0, 0)),
            pl.BlockSpec((1, 32), lambda i: (0, 0)),
        ],
        out_specs=[
            pl.BlockSpec((_T1, 32), lambda i: (i, 0)),
            pl.BlockSpec((_NB1, 32), lambda i: (i, 0)),
        ],
        out_shape=[
            jax.ShapeDtypeStruct((E, 32), jnp.float32),
            jax.ShapeDtypeStruct((N, 32), jnp.float32),
        ],
    )(ea, x, w1, b1.reshape(1, 32), w2, b2.reshape(1, 32))


def _sc_gather_mul_scatter(x32, ew1, src, dst):
    D = 32

    @functools.partial(
        pl.kernel,
        mesh=_sc_mesh(),
        out_type=jax.ShapeDtypeStruct((NC, N, D), jnp.float32),
        scratch_types=[
            pltpu.VMEM((CHUNK,), jnp.int32),
            pltpu.VMEM((CHUNK,), jnp.int32),
            pltpu.VMEM((CHUNK, D), jnp.float32),
            pltpu.VMEM((CHUNK, D), jnp.float32),
            pltpu.VMEM_SHARED((N, D), jnp.float32),
            pltpu.SemaphoreType.DMA,
        ],
        compiler_params=_SC_PARAMS,
    )
    def k(x32_hbm, ew_hbm, src_hbm, dst_hbm, out_hbm,
          src_v, dst_v, xrows_v, mrows_v, acc_shared, sem):
        cid = lax.axis_index("c")
        sid = lax.axis_index("s")
        wid = sid * NC + cid

        def zrow(r, carry):
            for c4 in range(D // 16):
                mrows_v[r, pl.ds(c4 * 16, 16)] = jnp.zeros((16,), jnp.float32)
            return carry

        lax.fori_loop(0, CHUNK, zrow, 0)
        base = sid * ROWS_PER_TILE
        off = 0
        for blk in (128, 128, 128, 128, 113):
            pltpu.sync_copy(mrows_v.at[pl.ds(0, blk)],
                            acc_shared.at[pl.ds(base + off, blk)])
            off += blk
        plsc.subcore_barrier()

        def body(j, carry):
            c = wid + NW * j

            @pl.when(c < NUM_CHUNKS)
            def _():
                pltpu.sync_copy(src_hbm.at[pl.ds(c * CHUNK, CHUNK)], src_v)
                pltpu.sync_copy(dst_hbm.at[pl.ds(c * CHUNK, CHUNK)], dst_v)
                pltpu.async_copy(x32_hbm.at[src_v], xrows_v, sem).wait()
                pltpu.sync_copy(ew_hbm.at[pl.ds(c * CHUNK, CHUNK)], mrows_v)

                def mulrow(r, carry2):
                    for c4 in range(D // 16):
                        sl = pl.ds(c4 * 16, 16)
                        mrows_v[r, sl] = mrows_v[r, sl] * xrows_v[r, sl]
                    return carry2

                lax.fori_loop(0, CHUNK, mulrow, 0)
                pltpu.sync_copy(mrows_v, acc_shared.at[dst_v], add=True)

            return carry

        lax.fori_loop(0, CPW, body, 0)
        plsc.subcore_barrier()
        pltpu.sync_copy(acc_shared.at[pl.ds(base, ROWS_PER_TILE)],
                        out_hbm.at[cid, pl.ds(base, ROWS_PER_TILE)])

    return k(x32, ew1, src, dst)


def _tc_msg2(ea, xj2, w1, b1, w2, b2):
    s2 = w2.reshape(64, 32, 64).transpose(2, 1, 0).reshape(64, 2048)
    b2t = b2.reshape(32, 64).T
    w1t = w1.T
    ea_t = ea.T
    ne = xj2.shape[0]

    def body(ea_ref, xj_ref, w1_ref, b1_ref, s2_ref, b2_ref, out_ref):
        h_t = jnp.maximum(
            jnp.dot(w1_ref[...], ea_ref[...],
                    preferred_element_type=jnp.float32)
            + b1_ref[...], 0.0)
        xj_t = xj_ref[...].T
        xh = jnp.concatenate(
            [h_t * xj_t[i:i + 1, :] for i in range(32)],
            axis=0)
        msg_t = (
            jnp.dot(s2_ref[...], xh, preferred_element_type=jnp.float32)
            + jnp.dot(b2_ref[...], xj_t,
                      preferred_element_type=jnp.float32))
        out_ref[...] = msg_t.T

    return pl.pallas_call(
        body,
        grid=(ne // _T2,),
        in_specs=[
            pl.BlockSpec((2, _T2), lambda i: (0, i)),
            pl.BlockSpec((_T2, 32), lambda i: (i, 0)),
            pl.BlockSpec((64, 2), lambda i: (0, 0)),
            pl.BlockSpec((64, 1), lambda i: (0, 0)),
            pl.BlockSpec((64, 2048), lambda i: (0, 0)),
            pl.BlockSpec((64, 32), lambda i: (0, 0)),
        ],
        out_specs=pl.BlockSpec((_T2, 64), lambda i: (i, 0)),
        out_shape=jax.ShapeDtypeStruct((ne, 64), jnp.float32),
    )(ea_t, xj2, w1t, b1.reshape(64, 1), s2, b2t)


def _elu(a):
    return jnp.where(a > 0, a, jnp.exp(jnp.minimum(a, 0.0)) - 1.0)


def _tc_h1(p, x, root, bias):
    def body(p_ref, x_ref, root_ref, b_ref, out_ref):
        a = p_ref[0] + p_ref[1] + x_ref[...] * root_ref[...] + b_ref[...]
        out_ref[...] = _elu(a)

    return pl.pallas_call(
        body,
        out_shape=jax.ShapeDtypeStruct((N, 32), jnp.float32),
    )(p, x, root, bias.reshape(1, 32))


def _tc_tail(p2a, p2b, h1, root2, bias2, batch_row,
             lin1_w, lin1_b, lin2_w, lin2_b):
    def body(pa_ref, pb_ref, h1_ref, root_ref, b_ref, batch_ref,
             w1_ref, b1_ref, w2_ref, b2_ref, out_ref):
        a = pa_ref[0] + pa_ref[1] + pb_ref[0] + pb_ref[1] \
            + jnp.dot(h1_ref[...], root_ref[...],
                      preferred_element_type=jnp.float32) + b_ref[...]
        h2 = _elu(a)
        gids = lax.broadcasted_iota(jnp.int32, (G, N), 0)
        oh = jnp.where(gids == batch_ref[...], 1.0, 0.0)
        s = jnp.dot(oh, h2, preferred_element_type=jnp.float32)
        cnt = jnp.sum(oh, axis=1, keepdims=True)
        pooled = s / jnp.maximum(cnt, 1.0)
        z = _elu(jnp.dot(pooled, w1_ref[...],
                         preferred_element_type=jnp.float32) + b1_ref[...])
        logits = jnp.dot(z, w2_ref[...],
                         preferred_element_type=jnp.float32) + b2_ref[...]
        m = jnp.max(logits, axis=1, keepdims=True)
        lse = jnp.log(jnp.sum(jnp.exp(logits - m), axis=1, keepdims=True)) + m
        out_ref[...] = logits - lse

    return pl.pallas_call(
        body,
        out_shape=jax.ShapeDtypeStruct((G, 10), jnp.float32),
    )(p2a, p2b, h1, root2, bias2.reshape(1, 64), batch_row,
      lin1_w, lin1_b.reshape(1, 128), lin2_w, lin2_b.reshape(1, 10))


def kernel(x, edge_index, edge_attr, batch,
           nn1_w1, nn1_b1, nn1_w2, nn1_b2, conv1_root, conv1_bias,
           nn2_w1, nn2_b1, nn2_w2, nn2_b2, conv2_root, conv2_bias,
           lin1_w, lin1_b, lin2_w, lin2_b):
    src = edge_index[0]
    dst = edge_index[1]

    ew1, x32 = _tc_ew1_x32(edge_attr, x, nn1_w1, nn1_b1, nn1_w2, nn1_b2)
    p1 = _sc_gather_mul_scatter(x32, ew1, src, dst)
    h1 = _tc_h1(p1, x, conv1_root, conv1_bias)

    eh = E // 2
    xj2a = _sc_gather(h1, src[:eh], 32)
    xj2b = _sc_gather(h1, src[eh:], 32)
    msg2a = _tc_msg2(edge_attr[:eh], xj2a,
                     nn2_w1, nn2_b1, nn2_w2, nn2_b2)
    msg2b = _tc_msg2(edge_attr[eh:], xj2b,
                     nn2_w1, nn2_b1, nn2_w2, nn2_b2)
    p2a = _sc_scatter_add(msg2a, dst[:eh], 64)
    p2b = _sc_scatter_add(msg2b, dst[eh:], 64)
    return _tc_tail(p2a, p2b, h1, conv2_root, conv2_bias,
                    batch.reshape(1, N).astype(jnp.int32),
                    lin1_w, lin1_b, lin2_w, lin2_b)


# speedup vs baseline: 3.5845x; 1.0647x over previous
import functools

import jax
import jax.numpy as jnp
from jax import lax
from jax.experimental import pallas as pl
from jax.experimental.pallas import tpu as pltpu
from jax.experimental.pallas import tpu_sc as plsc

E = 160000
N = 10000
G = 64
CHUNK = 128
NUM_CHUNKS = E // CHUNK
NC = 2
NS = 16
NW = NC * NS
CPW = (NUM_CHUNKS + NW - 1) // NW
ROWS_PER_TILE = N // NS


def _sc_mesh():
    return plsc.VectorSubcoreMesh(core_axis_name="c", subcore_axis_name="s")


_SC_PARAMS = pltpu.CompilerParams(use_tc_tiling_on_sc=False)


def _sc_gather(table, idx, D):
    ne = idx.shape[0]
    num_chunks = ne // CHUNK
    cpw = (num_chunks + NW - 1) // NW

    @functools.partial(
        pl.kernel,
        mesh=_sc_mesh(),
        out_type=jax.ShapeDtypeStruct((ne, D), jnp.float32),
        scratch_types=[
            pltpu.VMEM((CHUNK,), jnp.int32),
            pltpu.VMEM((CHUNK, D), jnp.float32),
            pltpu.SemaphoreType.DMA,
        ],
        compiler_params=_SC_PARAMS,
    )
    def k(table_hbm, idx_hbm, out_hbm, idx_v, rows_v, sem):
        cid = lax.axis_index("c")
        sid = lax.axis_index("s")
        wid = sid * NC + cid

        def body(j, carry):
            c = wid + NW * j

            @pl.when(c < num_chunks)
            def _():
                pltpu.sync_copy(idx_hbm.at[pl.ds(c * CHUNK, CHUNK)], idx_v)
                pltpu.async_copy(table_hbm.at[idx_v], rows_v, sem).wait()
                pltpu.sync_copy(rows_v, out_hbm.at[pl.ds(c * CHUNK, CHUNK)])

            return carry

        lax.fori_loop(0, cpw, body, 0)

    return k(table, idx)


def _sc_scatter_add(msg, dst, D):
    ne = msg.shape[0]
    num_chunks = ne // CHUNK
    cpw = (num_chunks + NW - 1) // NW

    @functools.partial(
        pl.kernel,
        mesh=_sc_mesh(),
        out_type=jax.ShapeDtypeStruct((NC, N, D), jnp.float32),
        scratch_types=[
            pltpu.VMEM((CHUNK,), jnp.int32),
            pltpu.VMEM((CHUNK, D), jnp.float32),
            pltpu.VMEM_SHARED((N, D), jnp.float32),
            pltpu.SemaphoreType.DMA,
        ],
        compiler_params=_SC_PARAMS,
    )
    def k(msg_hbm, dst_hbm, out_hbm, idx_v, rows_v, acc_shared, sem):
        cid = lax.axis_index("c")
        sid = lax.axis_index("s")
        wid = sid * NC + cid

        def zrow(r, carry):
            for c4 in range(D // 16):
                rows_v[r, pl.ds(c4 * 16, 16)] = jnp.zeros((16,), jnp.float32)
            return carry

        lax.fori_loop(0, CHUNK, zrow, 0)
        base = sid * ROWS_PER_TILE
        off = 0
        for blk in (128, 128, 128, 128, 113):
            pltpu.sync_copy(rows_v.at[pl.ds(0, blk)],
                            acc_shared.at[pl.ds(base + off, blk)])
            off += blk
        plsc.subcore_barrier()

        def body(j, carry):
            c = wid + NW * j

            @pl.when(c < num_chunks)
            def _():
                pltpu.sync_copy(dst_hbm.at[pl.ds(c * CHUNK, CHUNK)], idx_v)
                pltpu.sync_copy(msg_hbm.at[pl.ds(c * CHUNK, CHUNK)], rows_v)
                pltpu.sync_copy(rows_v, acc_shared.at[idx_v], add=True)

            return carry

        lax.fori_loop(0, cpw, body, 0)
        plsc.subcore_barrier()
        pltpu.sync_copy(acc_shared.at[pl.ds(base, ROWS_PER_TILE)],
                        out_hbm.at[cid, pl.ds(base, ROWS_PER_TILE)])

    return k(msg, dst)


_T1 = 3200
_NB1 = N // (E // _T1)
_T2 = 640


def _tc_ew1_x32(ea, x, w1, b1, w2, b2, with_x32):
    ne = ea.shape[0]
    grid = ne // _T1
    nb = N // grid

    def body_x(ea_ref, x_ref, w1_ref, b1_ref, w2_ref, b2_ref,
               ew_ref, x32_ref):
        ea_v = ea_ref[...]
        h = jnp.maximum(
            ea_v[:, 0:1] * w1_ref[0:1, :] + ea_v[:, 1:2] * w1_ref[1:2, :]
            + b1_ref[...], 0.0)
        ew_ref[...] = jnp.dot(h, w2_ref[...],
                              preferred_element_type=jnp.float32) \
            + b2_ref[...]
        x32_ref[...] = jnp.broadcast_to(x_ref[...], (nb, 32))

    def body_p(ea_ref, x_ref, w1_ref, b1_ref, w2_ref, b2_ref, ew_ref):
        ea_v = ea_ref[...]
        h = jnp.maximum(
            ea_v[:, 0:1] * w1_ref[0:1, :] + ea_v[:, 1:2] * w1_ref[1:2, :]
            + b1_ref[...], 0.0)
        ew_ref[...] = jnp.dot(h, w2_ref[...],
                              preferred_element_type=jnp.float32) \
            + b2_ref[...]

    out_specs = [pl.BlockSpec((_T1, 32), lambda i: (i, 0))]
    out_shape = [jax.ShapeDtypeStruct((ne, 32), jnp.float32)]
    if with_x32:
        out_specs.append(pl.BlockSpec((nb, 32), lambda i: (i, 0)))
        out_shape.append(jax.ShapeDtypeStruct((N, 32), jnp.float32))

    return pl.pallas_call(
        body_x if with_x32 else body_p,
        grid=(grid,),
        in_specs=[
            pl.BlockSpec((_T1, 2), lambda i: (i, 0)),
            pl.BlockSpec((nb, 1), lambda i: (i, 0)),
            pl.BlockSpec((2, 32), lambda i: (0, 0)),
            pl.BlockSpec((1, 32), lambda i: (0, 0)),
            pl.BlockSpec((32, 32), lambda i: (0, 0)),
            pl.BlockSpec((1, 32), lambda i: (0, 0)),
        ],
        out_specs=out_specs if with_x32 else out_specs[0],
        out_shape=out_shape if with_x32 else out_shape[0],
    )(ea, x, w1, b1.reshape(1, 32), w2, b2.reshape(1, 32))


def _sc_gather_mul_scatter(x32, ew1, src, dst):
    D = 32
    ne = ew1.shape[0]
    num_chunks = ne // CHUNK
    cpw = (num_chunks + NW - 1) // NW

    @functools.partial(
        pl.kernel,
        mesh=_sc_mesh(),
        out_type=jax.ShapeDtypeStruct((NC, N, D), jnp.float32),
        scratch_types=[
            pltpu.VMEM((CHUNK,), jnp.int32),
            pltpu.VMEM((CHUNK,), jnp.int32),
            pltpu.VMEM((CHUNK, D), jnp.float32),
            pltpu.VMEM((CHUNK, D), jnp.float32),
            pltpu.VMEM_SHARED((N, D), jnp.float32),
            pltpu.SemaphoreType.DMA,
        ],
        compiler_params=_SC_PARAMS,
    )
    def k(x32_hbm, ew_hbm, src_hbm, dst_hbm, out_hbm,
          src_v, dst_v, xrows_v, mrows_v, acc_shared, sem):
        cid = lax.axis_index("c")
        sid = lax.axis_index("s")
        wid = sid * NC + cid

        def zrow(r, carry):
            for c4 in range(D // 16):
                mrows_v[r, pl.ds(c4 * 16, 16)] = jnp.zeros((16,), jnp.float32)
            return carry

        lax.fori_loop(0, CHUNK, zrow, 0)
        base = sid * ROWS_PER_TILE
        off = 0
        for blk in (128, 128, 128, 128, 113):
            pltpu.sync_copy(mrows_v.at[pl.ds(0, blk)],
                            acc_shared.at[pl.ds(base + off, blk)])
            off += blk
        plsc.subcore_barrier()

        def body(j, carry):
            c = wid + NW * j

            @pl.when(c < num_chunks)
            def _():
                pltpu.sync_copy(src_hbm.at[pl.ds(c * CHUNK, CHUNK)], src_v)
                pltpu.sync_copy(dst_hbm.at[pl.ds(c * CHUNK, CHUNK)], dst_v)
                pltpu.async_copy(x32_hbm.at[src_v], xrows_v, sem).wait()
                pltpu.sync_copy(ew_hbm.at[pl.ds(c * CHUNK, CHUNK)], mrows_v)

                def mulrow(r, carry2):
                    for c4 in range(D // 16):
                        sl = pl.ds(c4 * 16, 16)
                        mrows_v[r, sl] = mrows_v[r, sl] * xrows_v[r, sl]
                    return carry2

                lax.fori_loop(0, CHUNK, mulrow, 0)
                pltpu.sync_copy(mrows_v, acc_shared.at[dst_v], add=True)

            return carry

        lax.fori_loop(0, cpw, body, 0)
        plsc.subcore_barrier()
        pltpu.sync_copy(acc_shared.at[pl.ds(base, ROWS_PER_TILE)],
                        out_hbm.at[cid, pl.ds(base, ROWS_PER_TILE)])

    return k(x32, ew1, src, dst)


def _tc_msg2(ea, xj2, w1, b1, w2, b2):
    s2 = w2.reshape(64, 32, 64).transpose(2, 1, 0).reshape(64, 2048)
    b2t = b2.reshape(32, 64).T
    w1t = w1.T
    ea_t = ea.T
    ne = xj2.shape[0]

    def body(ea_ref, xj_ref, w1_ref, b1_ref, s2_ref, b2_ref, out_ref):
        h_t = jnp.maximum(
            jnp.dot(w1_ref[...], ea_ref[...],
                    preferred_element_type=jnp.float32)
            + b1_ref[...], 0.0)
        xj_t = xj_ref[...].T
        xh = jnp.concatenate(
            [h_t * xj_t[i:i + 1, :] for i in range(32)],
            axis=0)
        msg_t = (
            jnp.dot(s2_ref[...], xh, preferred_element_type=jnp.float32)
            + jnp.dot(b2_ref[...], xj_t,
                      preferred_element_type=jnp.float32))
        out_ref[...] = msg_t.T

    return pl.pallas_call(
        body,
        grid=(ne // _T2,),
        in_specs=[
            pl.BlockSpec((2, _T2), lambda i: (0, i)),
            pl.BlockSpec((_T2, 32), lambda i: (i, 0)),
            pl.BlockSpec((64, 2), lambda i: (0, 0)),
            pl.BlockSpec((64, 1), lambda i: (0, 0)),
            pl.BlockSpec((64, 2048), lambda i: (0, 0)),
            pl.BlockSpec((64, 32), lambda i: (0, 0)),
        ],
        out_specs=pl.BlockSpec((_T2, 64), lambda i: (i, 0)),
        out_shape=jax.ShapeDtypeStruct((ne, 64), jnp.float32),
    )(ea_t, xj2, w1t, b1.reshape(64, 1), s2, b2t)


def _elu(a):
    return jnp.where(a > 0, a, jnp.exp(jnp.minimum(a, 0.0)) - 1.0)


def _tc_h1(pa, pb, x, root, bias):
    def body(pa_ref, pb_ref, x_ref, root_ref, b_ref, out_ref):
        a = pa_ref[0] + pa_ref[1] + pb_ref[0] + pb_ref[1] \
            + x_ref[...] * root_ref[...] + b_ref[...]
        out_ref[...] = _elu(a)

    return pl.pallas_call(
        body,
        out_shape=jax.ShapeDtypeStruct((N, 32), jnp.float32),
    )(pa, pb, x, root, bias.reshape(1, 32))


def _tc_tail(p2a, p2b, h1, root2, bias2, batch_row,
             lin1_w, lin1_b, lin2_w, lin2_b):
    def body(pa_ref, pb_ref, h1_ref, root_ref, b_ref, batch_ref,
             w1_ref, b1_ref, w2_ref, b2_ref, out_ref):
        a = pa_ref[0] + pa_ref[1] + pb_ref[0] + pb_ref[1] \
            + jnp.dot(h1_ref[...], root_ref[...],
                      preferred_element_type=jnp.float32) + b_ref[...]
        h2 = _elu(a)
        gids = lax.broadcasted_iota(jnp.int32, (G, N), 0)
        oh = jnp.where(gids == batch_ref[...], 1.0, 0.0)
        s = jnp.dot(oh, h2, preferred_element_type=jnp.float32)
        cnt = jnp.sum(oh, axis=1, keepdims=True)
        pooled = s / jnp.maximum(cnt, 1.0)
        z = _elu(jnp.dot(pooled, w1_ref[...],
                         preferred_element_type=jnp.float32) + b1_ref[...])
        logits = jnp.dot(z, w2_ref[...],
                         preferred_element_type=jnp.float32) + b2_ref[...]
        m = jnp.max(logits, axis=1, keepdims=True)
        lse = jnp.log(jnp.sum(jnp.exp(logits - m), axis=1, keepdims=True)) + m
        out_ref[...] = logits - lse

    return pl.pallas_call(
        body,
        out_shape=jax.ShapeDtypeStruct((G, 10), jnp.float32),
    )(p2a, p2b, h1, root2, bias2.reshape(1, 64), batch_row,
      lin1_w, lin1_b.reshape(1, 128), lin2_w, lin2_b.reshape(1, 10))


def kernel(x, edge_index, edge_attr, batch,
           nn1_w1, nn1_b1, nn1_w2, nn1_b2, conv1_root, conv1_bias,
           nn2_w1, nn2_b1, nn2_w2, nn2_b2, conv2_root, conv2_bias,
           lin1_w, lin1_b, lin2_w, lin2_b):
    src = edge_index[0]
    dst = edge_index[1]

    eh = E // 2

    ew1a, x32 = _tc_ew1_x32(edge_attr[:eh], x,
                            nn1_w1, nn1_b1, nn1_w2, nn1_b2, True)
    ew1b = _tc_ew1_x32(edge_attr[eh:], x,
                       nn1_w1, nn1_b1, nn1_w2, nn1_b2, False)
    p1a = _sc_gather_mul_scatter(x32, ew1a, src[:eh], dst[:eh])
    p1b = _sc_gather_mul_scatter(x32, ew1b, src[eh:], dst[eh:])
    h1 = _tc_h1(p1a, p1b, x, conv1_root, conv1_bias)

    xj2a = _sc_gather(h1, src[:eh], 32)
    xj2b = _sc_gather(h1, src[eh:], 32)
    msg2a = _tc_msg2(edge_attr[:eh], xj2a,
                     nn2_w1, nn2_b1, nn2_w2, nn2_b2)
    msg2b = _tc_msg2(edge_attr[eh:], xj2b,
                     nn2_w1, nn2_b1, nn2_w2, nn2_b2)
    p2a = _sc_scatter_add(msg2a, dst[:eh], 64)
    p2b = _sc_scatter_add(msg2b, dst[eh:], 64)
    return _tc_tail(p2a, p2b, h1, conv2_root, conv2_bias,
                    batch.reshape(1, N).astype(jnp.int32),
                    lin1_w, lin1_b, lin2_w, lin2_b)


# speedup vs baseline: 3.6784x; 1.0262x over previous
import functools

import jax
import jax.numpy as jnp
from jax import lax
from jax.experimental import pallas as pl
from jax.experimental.pallas import tpu as pltpu
from jax.experimental.pallas import tpu_sc as plsc

E = 160000
N = 10000
G = 64
CHUNK = 128
NUM_CHUNKS = E // CHUNK
NC = 2
NS = 16
NW = NC * NS
CPW = (NUM_CHUNKS + NW - 1) // NW
ROWS_PER_TILE = N // NS


def _sc_mesh():
    return plsc.VectorSubcoreMesh(core_axis_name="c", subcore_axis_name="s")


_SC_PARAMS = pltpu.CompilerParams(use_tc_tiling_on_sc=False)


def _sc_gather(table, idx, D):
    ne = idx.shape[0]
    num_chunks = ne // CHUNK
    cpw = (num_chunks + NW - 1) // NW

    @functools.partial(
        pl.kernel,
        mesh=_sc_mesh(),
        out_type=jax.ShapeDtypeStruct((ne, D), jnp.float32),
        scratch_types=[
            pltpu.VMEM((CHUNK,), jnp.int32),
            pltpu.VMEM((CHUNK, D), jnp.float32),
            pltpu.SemaphoreType.DMA,
        ],
        compiler_params=_SC_PARAMS,
    )
    def k(table_hbm, idx_hbm, out_hbm, idx_v, rows_v, sem):
        cid = lax.axis_index("c")
        sid = lax.axis_index("s")
        wid = sid * NC + cid

        def body(j, carry):
            c = wid + NW * j

            @pl.when(c < num_chunks)
            def _():
                pltpu.sync_copy(idx_hbm.at[pl.ds(c * CHUNK, CHUNK)], idx_v)
                pltpu.async_copy(table_hbm.at[idx_v], rows_v, sem).wait()
                pltpu.sync_copy(rows_v, out_hbm.at[pl.ds(c * CHUNK, CHUNK)])

            return carry

        lax.fori_loop(0, cpw, body, 0)

    return k(table, idx)


def _sc_scatter_add(msg, dst, D):
    ne = msg.shape[0]
    num_chunks = ne // CHUNK
    cpw = (num_chunks + NW - 1) // NW

    @functools.partial(
        pl.kernel,
        mesh=_sc_mesh(),
        out_type=jax.ShapeDtypeStruct((NC, N, D), jnp.float32),
        scratch_types=[
            pltpu.VMEM((CHUNK,), jnp.int32),
            pltpu.VMEM((CHUNK, D), jnp.float32),
            pltpu.VMEM_SHARED((N, D), jnp.float32),
            pltpu.SemaphoreType.DMA,
        ],
        compiler_params=_SC_PARAMS,
    )
    def k(msg_hbm, dst_hbm, out_hbm, idx_v, rows_v, acc_shared, sem):
        cid = lax.axis_index("c")
        sid = lax.axis_index("s")
        wid = sid * NC + cid

        def zrow(r, carry):
            for c4 in range(D // 16):
                rows_v[r, pl.ds(c4 * 16, 16)] = jnp.zeros((16,), jnp.float32)
            return carry

        lax.fori_loop(0, CHUNK, zrow, 0)
        base = sid * ROWS_PER_TILE
        off = 0
        for blk in (128, 128, 128, 128, 113):
            pltpu.sync_copy(rows_v.at[pl.ds(0, blk)],
                            acc_shared.at[pl.ds(base + off, blk)])
            off += blk
        plsc.subcore_barrier()

        def body(j, carry):
            c = wid + NW * j

            @pl.when(c < num_chunks)
            def _():
                pltpu.sync_copy(dst_hbm.at[pl.ds(c * CHUNK, CHUNK)], idx_v)
                pltpu.sync_copy(msg_hbm.at[pl.ds(c * CHUNK, CHUNK)], rows_v)
                pltpu.sync_copy(rows_v, acc_shared.at[idx_v], add=True)

            return carry

        lax.fori_loop(0, cpw, body, 0)
        plsc.subcore_barrier()
        pltpu.sync_copy(acc_shared.at[pl.ds(base, ROWS_PER_TILE)],
                        out_hbm.at[cid, pl.ds(base, ROWS_PER_TILE)])

    return k(msg, dst)


_T1 = 3200
_NB1 = N // (E // _T1)
_T2 = 640


def _tc_ew1_x32(ea, x, w1, b1, w2, b2, with_x32):
    ne = ea.shape[0]
    grid = ne // _T1
    nb = N // grid

    def body_x(ea_ref, x_ref, w1_ref, b1_ref, w2_ref, b2_ref,
               ew_ref, x32_ref):
        ea_v = ea_ref[...]
        h = jnp.maximum(
            ea_v[:, 0:1] * w1_ref[0:1, :] + ea_v[:, 1:2] * w1_ref[1:2, :]
            + b1_ref[...], 0.0)
        ew_ref[...] = jnp.dot(h, w2_ref[...],
                              preferred_element_type=jnp.float32) \
            + b2_ref[...]
        x32_ref[...] = jnp.broadcast_to(x_ref[...], (nb, 32))

    def body_p(ea_ref, x_ref, w1_ref, b1_ref, w2_ref, b2_ref, ew_ref):
        ea_v = ea_ref[...]
        h = jnp.maximum(
            ea_v[:, 0:1] * w1_ref[0:1, :] + ea_v[:, 1:2] * w1_ref[1:2, :]
            + b1_ref[...], 0.0)
        ew_ref[...] = jnp.dot(h, w2_ref[...],
                              preferred_element_type=jnp.float32) \
            + b2_ref[...]

    out_specs = [pl.BlockSpec((_T1, 32), lambda i: (i, 0))]
    out_shape = [jax.ShapeDtypeStruct((ne, 32), jnp.float32)]
    if with_x32:
        out_specs.append(pl.BlockSpec((nb, 32), lambda i: (i, 0)))
        out_shape.append(jax.ShapeDtypeStruct((N, 32), jnp.float32))

    return pl.pallas_call(
        body_x if with_x32 else body_p,
        grid=(grid,),
        in_specs=[
            pl.BlockSpec((_T1, 2), lambda i: (i, 0)),
            pl.BlockSpec((nb, 1), lambda i: (i, 0)),
            pl.BlockSpec((2, 32), lambda i: (0, 0)),
            pl.BlockSpec((1, 32), lambda i: (0, 0)),
            pl.BlockSpec((32, 32), lambda i: (0, 0)),
            pl.BlockSpec((1, 32), lambda i: (0, 0)),
        ],
        out_specs=out_specs if with_x32 else out_specs[0],
        out_shape=out_shape if with_x32 else out_shape[0],
    )(ea, x, w1, b1.reshape(1, 32), w2, b2.reshape(1, 32))


def _sc_gather_mul_scatter(x32, ew1, src, dst):
    D = 32
    ne = ew1.shape[0]
    num_chunks = ne // CHUNK
    cpw = (num_chunks + NW - 1) // NW

    @functools.partial(
        pl.kernel,
        mesh=_sc_mesh(),
        out_type=jax.ShapeDtypeStruct((NC, N, D), jnp.float32),
        scratch_types=[
            pltpu.VMEM((CHUNK,), jnp.int32),
            pltpu.VMEM((CHUNK,), jnp.int32),
            pltpu.VMEM((CHUNK, D), jnp.float32),
            pltpu.VMEM((CHUNK, D), jnp.float32),
            pltpu.VMEM_SHARED((N, D), jnp.float32),
            pltpu.SemaphoreType.DMA,
        ],
        compiler_params=_SC_PARAMS,
    )
    def k(x32_hbm, ew_hbm, src_hbm, dst_hbm, out_hbm,
          src_v, dst_v, xrows_v, mrows_v, acc_shared, sem):
        cid = lax.axis_index("c")
        sid = lax.axis_index("s")
        wid = sid * NC + cid

        def zrow(r, carry):
            for c4 in range(D // 16):
                mrows_v[r, pl.ds(c4 * 16, 16)] = jnp.zeros((16,), jnp.float32)
            return carry

        lax.fori_loop(0, CHUNK, zrow, 0)
        base = sid * ROWS_PER_TILE
        off = 0
        for blk in (128, 128, 128, 128, 113):
            pltpu.sync_copy(mrows_v.at[pl.ds(0, blk)],
                            acc_shared.at[pl.ds(base + off, blk)])
            off += blk
        plsc.subcore_barrier()

        def body(j, carry):
            c = wid + NW * j

            @pl.when(c < num_chunks)
            def _():
                pltpu.sync_copy(src_hbm.at[pl.ds(c * CHUNK, CHUNK)], src_v)
                pltpu.sync_copy(dst_hbm.at[pl.ds(c * CHUNK, CHUNK)], dst_v)
                pltpu.async_copy(x32_hbm.at[src_v], xrows_v, sem).wait()
                pltpu.sync_copy(ew_hbm.at[pl.ds(c * CHUNK, CHUNK)], mrows_v)

                def mulrow(r, carry2):
                    for c4 in range(D // 16):
                        sl = pl.ds(c4 * 16, 16)
                        mrows_v[r, sl] = mrows_v[r, sl] * xrows_v[r, sl]
                    return carry2

                lax.fori_loop(0, CHUNK, mulrow, 0)
                pltpu.sync_copy(mrows_v, acc_shared.at[dst_v], add=True)

            return carry

        lax.fori_loop(0, cpw, body, 0)
        plsc.subcore_barrier()
        pltpu.sync_copy(acc_shared.at[pl.ds(base, ROWS_PER_TILE)],
                        out_hbm.at[cid, pl.ds(base, ROWS_PER_TILE)])

    return k(x32, ew1, src, dst)


def _tc_msg2(ea, xj2, w1, b1, w2, b2):
    s2 = w2.reshape(64, 32, 64).transpose(2, 1, 0).reshape(64, 2048)
    b2t = b2.reshape(32, 64).T
    w1t = w1.T
    ea_t = ea.T
    ne = xj2.shape[0]

    def body(ea_ref, xj_ref, w1_ref, b1_ref, s2_ref, b2_ref, out_ref):
        h_t = jnp.maximum(
            jnp.dot(w1_ref[...], ea_ref[...],
                    preferred_element_type=jnp.float32)
            + b1_ref[...], 0.0)
        xj_t = xj_ref[...].T
        xh = jnp.concatenate(
            [h_t * xj_t[i:i + 1, :] for i in range(32)],
            axis=0)
        msg_t = (
            jnp.dot(s2_ref[...], xh, preferred_element_type=jnp.float32)
            + jnp.dot(b2_ref[...], xj_t,
                      preferred_element_type=jnp.float32))
        out_ref[...] = msg_t.T

    return pl.pallas_call(
        body,
        grid=(ne // _T2,),
        in_specs=[
            pl.BlockSpec((2, _T2), lambda i: (0, i)),
            pl.BlockSpec((_T2, 32), lambda i: (i, 0)),
            pl.BlockSpec((64, 2), lambda i: (0, 0)),
            pl.BlockSpec((64, 1), lambda i: (0, 0)),
            pl.BlockSpec((64, 2048), lambda i: (0, 0)),
            pl.BlockSpec((64, 32), lambda i: (0, 0)),
        ],
        out_specs=pl.BlockSpec((_T2, 64), lambda i: (i, 0)),
        out_shape=jax.ShapeDtypeStruct((ne, 64), jnp.float32),
    )(ea_t, xj2, w1t, b1.reshape(64, 1), s2, b2t)


def _elu(a):
    return jnp.where(a > 0, a, jnp.exp(jnp.minimum(a, 0.0)) - 1.0)


def _tc_h1(pa, pb, x, root, bias):
    def body(pa_ref, pb_ref, x_ref, root_ref, b_ref, out_ref):
        a = pa_ref[0] + pa_ref[1] + pb_ref[0] + pb_ref[1] \
            + x_ref[...] * root_ref[...] + b_ref[...]
        out_ref[...] = _elu(a)

    return pl.pallas_call(
        body,
        out_shape=jax.ShapeDtypeStruct((N, 32), jnp.float32),
    )(pa, pb, x, root, bias.reshape(1, 32))


def _tc_tail(p2a, p2b, p2c, h1, root2, bias2, batch_row,
             lin1_w, lin1_b, lin2_w, lin2_b):
    def body(pa_ref, pb_ref, pc_ref, h1_ref, root_ref, b_ref, batch_ref,
             w1_ref, b1_ref, w2_ref, b2_ref, out_ref):
        a = pa_ref[0] + pa_ref[1] + pb_ref[0] + pb_ref[1] \
            + pc_ref[0] + pc_ref[1] \
            + jnp.dot(h1_ref[...], root_ref[...],
                      preferred_element_type=jnp.float32) + b_ref[...]
        h2 = _elu(a)
        gids = lax.broadcasted_iota(jnp.int32, (G, N), 0)
        oh = jnp.where(gids == batch_ref[...], 1.0, 0.0)
        s = jnp.dot(oh, h2, preferred_element_type=jnp.float32)
        cnt = jnp.sum(oh, axis=1, keepdims=True)
        pooled = s / jnp.maximum(cnt, 1.0)
        z = _elu(jnp.dot(pooled, w1_ref[...],
                         preferred_element_type=jnp.float32) + b1_ref[...])
        logits = jnp.dot(z, w2_ref[...],
                         preferred_element_type=jnp.float32) + b2_ref[...]
        m = jnp.max(logits, axis=1, keepdims=True)
        lse = jnp.log(jnp.sum(jnp.exp(logits - m), axis=1, keepdims=True)) + m
        out_ref[...] = logits - lse

    return pl.pallas_call(
        body,
        out_shape=jax.ShapeDtypeStruct((G, 10), jnp.float32),
    )(p2a, p2b, p2c, h1, root2, bias2.reshape(1, 64), batch_row,
      lin1_w, lin1_b.reshape(1, 128), lin2_w, lin2_b.reshape(1, 10))


def kernel(x, edge_index, edge_attr, batch,
           nn1_w1, nn1_b1, nn1_w2, nn1_b2, conv1_root, conv1_bias,
           nn2_w1, nn2_b1, nn2_w2, nn2_b2, conv2_root, conv2_bias,
           lin1_w, lin1_b, lin2_w, lin2_b):
    src = edge_index[0]
    dst = edge_index[1]

    eh = E // 2

    ew1a, x32 = _tc_ew1_x32(edge_attr[:eh], x,
                            nn1_w1, nn1_b1, nn1_w2, nn1_b2, True)
    ew1b = _tc_ew1_x32(edge_attr[eh:], x,
                       nn1_w1, nn1_b1, nn1_w2, nn1_b2, False)
    p1a = _sc_gather_mul_scatter(x32, ew1a, src[:eh], dst[:eh])
    p1b = _sc_gather_mul_scatter(x32, ew1b, src[eh:], dst[eh:])
    h1 = _tc_h1(p1a, p1b, x, conv1_root, conv1_bias)

    e0, e1 = 38400, 121600
    parts = []
    for lo, hi in ((0, e0), (e0, e1), (e1, E)):
        xj2 = _sc_gather(h1, src[lo:hi], 32)
        msg2 = _tc_msg2(edge_attr[lo:hi], xj2,
                        nn2_w1, nn2_b1, nn2_w2, nn2_b2)
        parts.append(_sc_scatter_add(msg2, dst[lo:hi], 64))
    return _tc_tail(parts[0], parts[1], parts[2], h1,
                    conv2_root, conv2_bias,
                    batch.reshape(1, N).astype(jnp.int32),
                    lin1_w, lin1_b, lin2_w, lin2_b)
